# baseline (speedup 1.0000x reference)
import functools

import jax
import jax.numpy as jnp
from jax import lax
from jax.experimental import pallas as pl
from jax.experimental.pallas import tpu as pltpu
from jax.experimental.pallas import tpu_sc as plsc

D = 512
NW = 32
CHUNK = 56
NBUF = 4
N_CHUNKS = 56
B_PER_W = CHUNK * N_CHUNKS
B_PAD = B_PER_W * NW
N_STEPS = N_CHUNKS // NBUF

_mesh = plsc.VectorSubcoreMesh(core_axis_name="c", subcore_axis_name="s")


@functools.partial(
    pl.kernel,
    mesh=_mesh,
    out_type=jax.ShapeDtypeStruct((B_PAD, D), jnp.float32),
    scratch_types=[
        pltpu.VMEM((B_PER_W,), jnp.int32),
        *[pltpu.VMEM((CHUNK, D), jnp.float32) for _ in range(NBUF)],
        *[pltpu.SemaphoreType.DMA for _ in range(2 * NBUF)],
    ],
)
def _gather_kernel(table_hbm, idx_hbm, out_hbm, idx_v, *rest):
    bufs = rest[:NBUF]
    gsems = rest[NBUF : 2 * NBUF]
    ssems = rest[2 * NBUF :]

    wid = lax.axis_index("s") * 2 + lax.axis_index("c")
    base = wid * B_PER_W

    pltpu.sync_copy(idx_hbm.at[pl.ds(base, B_PER_W)], idx_v)

    def start_gather(j, b):
        off = pl.multiple_of(j * CHUNK, 8)
        pltpu.async_copy(
            table_hbm.at[idx_v.at[pl.ds(off, CHUNK)]], bufs[b], gsems[b]
        )

    def start_scatter(j, b):
        off = pl.multiple_of(j * CHUNK, 8)
        pltpu.async_copy(bufs[b], out_hbm.at[pl.ds(base + off, CHUNK)], ssems[b])

    def wait_gather(b):
        pltpu.make_async_copy(
            table_hbm.at[pl.ds(0, CHUNK)], bufs[b], gsems[b]
        ).wait()

    def wait_scatter(b):
        pltpu.make_async_copy(
            bufs[b], out_hbm.at[pl.ds(base, CHUNK)], ssems[b]
        ).wait()

    for b in range(NBUF):
        start_gather(b, b)

    def body(p, carry):
        j = p * NBUF
        for b in range(NBUF):
            wait_gather(b)
            start_scatter(j + b, b)
        for b in range(NBUF):
            wait_scatter(b)

            @pl.when(p < N_STEPS - 1)
            def _():
                start_gather(j + NBUF + b, b)

        return carry

    lax.fori_loop(0, N_STEPS, body, 0)


def kernel(x_feats, upsample_indices):
    idx = upsample_indices.astype(jnp.int32)
    b = idx.shape[0]
    idx_pad = jnp.pad(idx, (0, B_PAD - b))
    out = _gather_kernel(x_feats, idx_pad)
    return out[:b]


# speedup vs baseline: 1.8884x; 1.8884x over previous
import functools

import jax
import jax.numpy as jnp
from jax import lax
from jax.experimental import pallas as pl
from jax.experimental.pallas import tpu as pltpu
from jax.experimental.pallas import tpu_sc as plsc

D = 512
NW = 32
CHUNK = 112
B_PER_W = 3136
B_OUT = 100000
TAIL_W = NW - 1
TAIL_FULL = 24
TAIL_REM = 96
B_IDX_PAD = B_PER_W * NW

_mesh = plsc.VectorSubcoreMesh(core_axis_name="c", subcore_axis_name="s")


@functools.partial(
    pl.kernel,
    mesh=_mesh,
    out_type=jax.ShapeDtypeStruct((B_OUT, D), jnp.float32),
    scratch_types=[
        pltpu.VMEM((B_PER_W,), jnp.int32),
        pltpu.VMEM((CHUNK, D), jnp.float32),
        pltpu.VMEM((CHUNK, D), jnp.float32),
        pltpu.SemaphoreType.DMA,
        pltpu.SemaphoreType.DMA,
        pltpu.SemaphoreType.DMA,
        pltpu.SemaphoreType.DMA,
    ],
)
def _gather_kernel(
    table_hbm, idx_hbm, out_hbm, idx_v, buf0, buf1, gsem0, gsem1, ssem0, ssem1
):
    bufs = (buf0, buf1)
    gsems = (gsem0, gsem1)
    ssems = (ssem0, ssem1)

    wid = lax.axis_index("s") * 2 + lax.axis_index("c")
    base = wid * B_PER_W
    n_pairs = jnp.where(wid == TAIL_W, TAIL_FULL // 2, 28 // 2)

    pltpu.sync_copy(idx_hbm.at[pl.ds(base, B_PER_W)], idx_v)

    def start_gather(j, b):
        off = pl.multiple_of(j * CHUNK, 8)
        pltpu.async_copy(
            table_hbm.at[idx_v.at[pl.ds(off, CHUNK)]], bufs[b], gsems[b]
        )

    def start_scatter(j, b):
        off = pl.multiple_of(j * CHUNK, 8)
        pltpu.async_copy(bufs[b], out_hbm.at[pl.ds(base + off, CHUNK)], ssems[b])

    def wait_gather(b):
        pltpu.make_async_copy(
            table_hbm.at[pl.ds(0, CHUNK)], bufs[b], gsems[b]
        ).wait()

    def wait_scatter(b):
        pltpu.make_async_copy(
            bufs[b], out_hbm.at[pl.ds(base, CHUNK)], ssems[b]
        ).wait()

    start_gather(0, 0)
    start_gather(1, 1)

    def body(p, carry):
        j = p * 2
        for b in range(2):
            wait_gather(b)
            start_scatter(j + b, b)
        for b in range(2):
            wait_scatter(b)

            @pl.when(p < n_pairs - 1)
            def _():
                start_gather(j + 2 + b, b)

        return carry

    lax.fori_loop(0, n_pairs, body, 0)

    @pl.when(wid == TAIL_W)
    def _():
        off = TAIL_FULL * CHUNK
        pltpu.async_copy(
            table_hbm.at[idx_v.at[pl.ds(off, TAIL_REM)]],
            buf0.at[pl.ds(0, TAIL_REM)],
            gsem0,
        ).wait()
        pltpu.async_copy(
            buf0.at[pl.ds(0, TAIL_REM)],
            out_hbm.at[pl.ds(base + off, TAIL_REM)],
            ssem0,
        ).wait()


def kernel(x_feats, upsample_indices):
    idx = upsample_indices.astype(jnp.int32)
    b = idx.shape[0]
    idx_pad = jnp.pad(idx, (0, B_IDX_PAD - b))
    return _gather_kernel(x_feats, idx_pad)


# speedup vs baseline: 1.9017x; 1.0070x over previous
import functools

import jax
import jax.numpy as jnp
from jax import lax
from jax.experimental import pallas as pl
from jax.experimental.pallas import tpu as pltpu
from jax.experimental.pallas import tpu_sc as plsc

D = 512
NW = 32
CHUNK = 112
B_PER_W = 3136
B_OUT = 100000
TAIL_W = NW - 1
TAIL_FULL = 24
TAIL_REM = 96
TAIL_ROWS = TAIL_FULL * CHUNK + TAIL_REM

_mesh = plsc.VectorSubcoreMesh(core_axis_name="c", subcore_axis_name="s")


@functools.partial(
    pl.kernel,
    mesh=_mesh,
    out_type=jax.ShapeDtypeStruct((B_OUT, D), jnp.float32),
    scratch_types=[
        pltpu.VMEM((B_PER_W,), jnp.int32),
        pltpu.VMEM((CHUNK, D), jnp.float32),
        pltpu.VMEM((CHUNK, D), jnp.float32),
        pltpu.SemaphoreType.DMA,
        pltpu.SemaphoreType.DMA,
        pltpu.SemaphoreType.DMA,
        pltpu.SemaphoreType.DMA,
    ],
)
def _gather_kernel(
    table_hbm, idx_hbm, out_hbm, idx_v, buf0, buf1, gsem0, gsem1, ssem0, ssem1
):
    bufs = (buf0, buf1)
    gsems = (gsem0, gsem1)
    ssems = (ssem0, ssem1)

    wid = lax.axis_index("s") * 2 + lax.axis_index("c")
    base = wid * B_PER_W
    n_pairs = jnp.where(wid == TAIL_W, TAIL_FULL // 2, 28 // 2)

    @pl.when(wid != TAIL_W)
    def _():
        pltpu.sync_copy(idx_hbm.at[pl.ds(base, B_PER_W)], idx_v)

    @pl.when(wid == TAIL_W)
    def _():
        pltpu.sync_copy(
            idx_hbm.at[pl.ds(base, TAIL_ROWS)], idx_v.at[pl.ds(0, TAIL_ROWS)]
        )

    def start_gather(j, b):
        off = pl.multiple_of(j * CHUNK, 8)
        pltpu.async_copy(
            table_hbm.at[idx_v.at[pl.ds(off, CHUNK)]], bufs[b], gsems[b]
        )

    def start_scatter(j, b):
        off = pl.multiple_of(j * CHUNK, 8)
        pltpu.async_copy(bufs[b], out_hbm.at[pl.ds(base + off, CHUNK)], ssems[b])

    def wait_gather(b):
        pltpu.make_async_copy(
            table_hbm.at[pl.ds(0, CHUNK)], bufs[b], gsems[b]
        ).wait()

    def wait_scatter(b):
        pltpu.make_async_copy(
            bufs[b], out_hbm.at[pl.ds(base, CHUNK)], ssems[b]
        ).wait()

    start_gather(0, 0)
    start_gather(1, 1)

    def body(p, carry):
        j = p * 2
        for b in range(2):
            wait_gather(b)
            start_scatter(j + b, b)
        for b in range(2):
            wait_scatter(b)

            @pl.when(p < n_pairs - 1)
            def _():
                start_gather(j + 2 + b, b)

        return carry

    lax.fori_loop(0, n_pairs, body, 0)

    @pl.when(wid == TAIL_W)
    def _():
        off = TAIL_FULL * CHUNK
        pltpu.async_copy(
            table_hbm.at[idx_v.at[pl.ds(off, TAIL_REM)]],
            buf0.at[pl.ds(0, TAIL_REM)],
            gsem0,
        ).wait()
        pltpu.async_copy(
            buf0.at[pl.ds(0, TAIL_REM)],
            out_hbm.at[pl.ds(base + off, TAIL_REM)],
            ssem0,
        ).wait()


def kernel(x_feats, upsample_indices):
    idx = upsample_indices.astype(jnp.int32)
    return _gather_kernel(x_feats, idx)
